# baseline (speedup 1.0000x reference)
import functools

import jax
import jax.numpy as jnp
from jax import lax
from jax.experimental import pallas as pl
from jax.experimental.pallas import tpu as pltpu
from jax.experimental.pallas import tpu_sc as plsc

def _make_sc_gather(V, D, B):
    info = plsc.get_sparse_core_info()
    _NC = 1
    _NS = 8
    _NW = _NC * _NS
    b_per_w = B // _NW
    mesh = plsc.VectorSubcoreMesh(
        core_axis_name="c", subcore_axis_name="s", num_cores=_NC, num_subcores=_NS)

    @functools.partial(
        pl.kernel,
        mesh=mesh,
        out_type=jax.ShapeDtypeStruct((B, D), jnp.float32),
        scratch_types=[
            pltpu.VMEM((b_per_w,), jnp.int32),
            pltpu.VMEM((b_per_w, D), jnp.float32),
            pltpu.SemaphoreType.DMA,
        ],
    )
    def gather(table_hbm, idx_hbm, out_hbm, idx_v, rows_v, sem):
        wid = lax.axis_index("s") * _NC + lax.axis_index("c")
        base = wid * b_per_w
        pltpu.sync_copy(idx_hbm.at[pl.ds(base, b_per_w)], idx_v)
        pltpu.async_copy(table_hbm.at[idx_v], rows_v, sem).wait()
        pltpu.sync_copy(rows_v, out_hbm.at[pl.ds(base, b_per_w)])

    return gather


def _tc_body(gate_ref, sp_ref, gamma_ref, beta_ref, x_ref, o_ref):
    sp = sp_ref[...]
    mean = jnp.mean(sp, axis=-1, keepdims=True)
    cen = sp - mean
    var = jnp.mean(cen * cen, axis=-1, keepdims=True)
    norm = cen * lax.rsqrt(var + 1e-5)
    norm = norm * gamma_ref[...] + beta_ref[...]
    g = jax.nn.sigmoid(gate_ref[0])
    o_ref[...] = x_ref[...] + g * norm[:, None, :]


def _tc_fused(x, sp_emb, gamma, beta, gate):
    B, L, D = x.shape
    BB = 128
    grid = (B // BB,)
    return pl.pallas_call(
        _tc_body,
        grid=grid,
        in_specs=[
            pl.BlockSpec(memory_space=pltpu.SMEM),
            pl.BlockSpec((BB, D), lambda i: (i, 0)),
            pl.BlockSpec((1, D), lambda i: (0, 0)),
            pl.BlockSpec((1, D), lambda i: (0, 0)),
            pl.BlockSpec((BB, L, D), lambda i: (i, 0, 0)),
        ],
        out_specs=pl.BlockSpec((BB, L, D), lambda i: (i, 0, 0)),
        out_shape=jax.ShapeDtypeStruct((B, L, D), x.dtype),
        compiler_params=pltpu.CompilerParams(
            vmem_limit_bytes=128 * 1024 * 1024,
        ),
    )(gate, sp_emb, gamma, beta, x)


def kernel(x, species_idx, table, gamma, beta, gate):
    B, L, D = x.shape
    V = table.shape[0]
    idx = species_idx.astype(jnp.int32)
    sp_emb = _make_sc_gather(V, D, B)(table, idx)
    return _tc_fused(
        x,
        sp_emb,
        gamma.reshape(1, D),
        beta.reshape(1, D),
        jnp.asarray(gate, jnp.float32).reshape(1),
    )


# speedup vs baseline: 1.0089x; 1.0089x over previous
import functools

import jax
import jax.numpy as jnp
from jax import lax
from jax.experimental import pallas as pl
from jax.experimental.pallas import tpu as pltpu
from jax.experimental.pallas import tpu_sc as plsc

def _make_sc_gather(V, D, B):
    info = plsc.get_sparse_core_info()
    _NC = 1
    _NS = info.num_subcores
    _NW = _NC * _NS
    b_per_w = B // _NW
    mesh = plsc.VectorSubcoreMesh(
        core_axis_name="c", subcore_axis_name="s", num_cores=_NC, num_subcores=_NS)

    half = b_per_w // 2

    @functools.partial(
        pl.kernel,
        mesh=mesh,
        out_type=jax.ShapeDtypeStruct((B, D), jnp.float32),
        scratch_types=[
            pltpu.VMEM((b_per_w,), jnp.int32),
            pltpu.VMEM((half, D), jnp.float32),
            pltpu.VMEM((half, D), jnp.float32),
            pltpu.SemaphoreType.DMA,
            pltpu.SemaphoreType.DMA,
        ],
    )
    def gather(table_hbm, idx_hbm, out_hbm, idx_v, rows_a, rows_b, sem_a, sem_b):
        wid = lax.axis_index("s") * _NC + lax.axis_index("c")
        base = wid * b_per_w
        pltpu.sync_copy(idx_hbm.at[pl.ds(base, b_per_w)], idx_v)
        ca = pltpu.async_copy(table_hbm.at[idx_v.at[pl.ds(0, half)]], rows_a, sem_a)
        cb = pltpu.async_copy(table_hbm.at[idx_v.at[pl.ds(half, half)]], rows_b, sem_b)
        ca.wait()
        pltpu.sync_copy(rows_a, out_hbm.at[pl.ds(base, half)])
        cb.wait()
        pltpu.sync_copy(rows_b, out_hbm.at[pl.ds(base + half, half)])

    return gather


def _tc_body(gate_ref, sp_ref, gamma_ref, beta_ref, x_ref, o_ref):
    sp = sp_ref[...]
    mean = jnp.mean(sp, axis=-1, keepdims=True)
    cen = sp - mean
    var = jnp.mean(cen * cen, axis=-1, keepdims=True)
    norm = cen * lax.rsqrt(var + 1e-5)
    norm = norm * gamma_ref[...] + beta_ref[...]
    g = jax.nn.sigmoid(gate_ref[0])
    o_ref[...] = x_ref[...] + g * norm[:, None, :]


def _tc_fused(x, sp_emb, gamma, beta, gate):
    B, L, D = x.shape
    BB = 128
    grid = (B // BB,)
    return pl.pallas_call(
        _tc_body,
        grid=grid,
        in_specs=[
            pl.BlockSpec(memory_space=pltpu.SMEM),
            pl.BlockSpec((BB, D), lambda i: (i, 0)),
            pl.BlockSpec((1, D), lambda i: (0, 0)),
            pl.BlockSpec((1, D), lambda i: (0, 0)),
            pl.BlockSpec((BB, L, D), lambda i: (i, 0, 0)),
        ],
        out_specs=pl.BlockSpec((BB, L, D), lambda i: (i, 0, 0)),
        out_shape=jax.ShapeDtypeStruct((B, L, D), x.dtype),
        compiler_params=pltpu.CompilerParams(
            vmem_limit_bytes=128 * 1024 * 1024,
        ),
    )(gate, sp_emb, gamma, beta, x)


def kernel(x, species_idx, table, gamma, beta, gate):
    B, L, D = x.shape
    V = table.shape[0]
    idx = species_idx.astype(jnp.int32)
    sp_emb = _make_sc_gather(V, D, B)(table, idx)
    return _tc_fused(
        x,
        sp_emb,
        gamma.reshape(1, D),
        beta.reshape(1, D),
        jnp.asarray(gate, jnp.float32).reshape(1),
    )
